# baseline (speedup 1.0000x reference)
import jax
import jax.numpy as jnp
from jax.experimental import pallas as pl
from jax.experimental.pallas import tpu as pltpu

_BM = 400
_BMM = 1000


def _mm_kernel(a_ref, w_ref, o_ref):
    o_ref[...] = jnp.dot(a_ref[...], w_ref[...],
                         preferred_element_type=jnp.float32)


def _layer1_kernel(adj_ref, s_ref, b_ref, o_ref):
    acc = jnp.dot(adj_ref[...], s_ref[...],
                  preferred_element_type=jnp.float32)
    o_ref[...] = jnp.maximum(acc + b_ref[...], 0.0)


def _layer2_kernel(adj_ref, s_ref, b_ref, w3_ref, b3_ref, emb_ref, sc_ref):
    acc = jnp.dot(adj_ref[...], s_ref[...],
                  preferred_element_type=jnp.float32)
    e = jnp.maximum(acc + b_ref[...], 0.0)
    emb_ref[...] = e
    sc_ref[...] = jnp.sum(e * w3_ref[...], axis=1, keepdims=True) + b3_ref[...]


def _small_mm(a, w):
    n, f = a.shape
    return pl.pallas_call(
        _mm_kernel,
        grid=(n // _BMM,),
        in_specs=[
            pl.BlockSpec((_BMM, f), lambda i: (i, 0)),
            pl.BlockSpec((f, w.shape[1]), lambda i: (0, 0)),
        ],
        out_specs=pl.BlockSpec((_BMM, w.shape[1]), lambda i: (i, 0)),
        out_shape=jax.ShapeDtypeStruct((n, w.shape[1]), jnp.float32),
    )(a, w)


def kernel(x, adj, W1, b1, W2, b2, W3, b3):
    n, f = x.shape
    nh = W1.shape[1]
    b1r = b1.reshape(1, nh)
    b2r = b2.reshape(1, nh)
    b3r = b3.reshape(1, 1)

    s1 = _small_mm(x, W1)

    h = pl.pallas_call(
        _layer1_kernel,
        grid=(n // _BM,),
        in_specs=[
            pl.BlockSpec((_BM, n), lambda i: (i, 0)),
            pl.BlockSpec((n, nh), lambda i: (0, 0)),
            pl.BlockSpec((1, nh), lambda i: (0, 0)),
        ],
        out_specs=pl.BlockSpec((_BM, nh), lambda i: (i, 0)),
        out_shape=jax.ShapeDtypeStruct((n, nh), jnp.float32),
        compiler_params=pltpu.CompilerParams(
            dimension_semantics=("arbitrary",)),
    )(adj, s1, b1r)

    s2 = _small_mm(h, W2)

    emb, score = pl.pallas_call(
        _layer2_kernel,
        grid=(n // _BM,),
        in_specs=[
            pl.BlockSpec((_BM, n), lambda i: (i, 0)),
            pl.BlockSpec((n, nh), lambda i: (0, 0)),
            pl.BlockSpec((1, nh), lambda i: (0, 0)),
            pl.BlockSpec((1, nh), lambda i: (0, 0)),
            pl.BlockSpec((1, 1), lambda i: (0, 0)),
        ],
        out_specs=[
            pl.BlockSpec((_BM, nh), lambda i: (i, 0)),
            pl.BlockSpec((_BM, 1), lambda i: (i, 0)),
        ],
        out_shape=[
            jax.ShapeDtypeStruct((n, nh), jnp.float32),
            jax.ShapeDtypeStruct((n, 1), jnp.float32),
        ],
        compiler_params=pltpu.CompilerParams(
            dimension_semantics=("arbitrary",)),
    )(adj, s2, b2r, W3, b3r)

    return (score, emb)
